# baseline (speedup 1.0000x reference)
import jax
import jax.numpy as jnp
import numpy as np
from jax.experimental import pallas as pl
from jax.experimental.pallas import tpu as pltpu

_IDX = np.array(
    [3, 17, 29, 42, 56, 61, 73, 88, 91, 104, 111, 123, 130, 142, 150,
     158, 163, 171, 180, 187, 195, 7, 12, 25, 33, 47, 52, 66, 79, 83,
     96, 101, 115, 127, 135, 146, 153, 167, 174, 182, 190, 199, 5, 19,
     38, 59, 70, 99, 119, 139],
    dtype=np.int32,
)
_K = _IDX.shape[0]


def _copy_body(idx_ref, x_ref, o_ref):
    o_ref[...] = x_ref[...]


def kernel(x):
    B, R, F = x.shape
    xt = jnp.transpose(x, (1, 2, 0))
    idx = jnp.asarray(_IDX)

    out_t = pl.pallas_call(
        _copy_body,
        grid_spec=pltpu.PrefetchScalarGridSpec(
            num_scalar_prefetch=1,
            grid=(_K, 2),
            in_specs=[
                pl.BlockSpec((1, F, B // 2), lambda j, b, idx_ref: (idx_ref[j], 0, b)),
            ],
            out_specs=pl.BlockSpec((1, F, B // 2), lambda j, b, idx_ref: (j, 0, b)),
        ),
        out_shape=jax.ShapeDtypeStruct((_K, F, B), x.dtype),
    )(idx, xt)
    return out_t.transpose(2, 0, 1)
